# Initial kernel scaffold; baseline (speedup 1.0000x reference)
import jax
import jax.numpy as jnp
from jax.experimental import pallas as pl


def kernel(x, w_stack):
    raise NotImplementedError("write your pallas kernel here")


# speedup vs baseline: 33.9599x; 33.9599x over previous
import jax
import jax.numpy as jnp
from jax import lax
from jax.experimental import pallas as pl
from jax.experimental.pallas import tpu as pltpu

_HID = 7
_NL = 10
_SLOPE = 0.01
_LANES = 128
_TR = 512
_CH = 32


def _leaky(a):
    return jnp.maximum(a, _SLOPE * a)


def _rtne_bf16(a):
    u = lax.bitcast_convert_type(a, jnp.uint32)
    r = (u + jnp.uint32(0x7FFF) + ((u >> 16) & jnp.uint32(1))) & jnp.uint32(
        0xFFFF0000)
    return lax.bitcast_convert_type(r, jnp.float32)


def _mlp_kernel(w_ref, x_ref, o_ref):
    def chunk(c, carry):
        r0 = c * _CH
        x0 = _rtne_bf16(x_ref[0, pl.ds(r0, _CH), :])
        x1 = _rtne_bf16(x_ref[1, pl.ds(r0, _CH), :])
        h = []
        for o in range(_HID):
            a = x0 * w_ref[0, 0, o] + x1 * w_ref[0, 1, o]
            h.append(_leaky(a))
        for l in range(1, _NL - 1):
            hr = [_rtne_bf16(v) for v in h]
            hn = []
            for o in range(_HID):
                a = hr[0] * w_ref[l, 0, o]
                for i in range(1, _HID):
                    a = a + hr[i] * w_ref[l, i, o]
                hn.append(_leaky(a))
            h = hn
        hr = [_rtne_bf16(v) for v in h]
        a = hr[0] * w_ref[_NL - 1, 0, 0]
        for i in range(1, _HID):
            a = a + hr[i] * w_ref[_NL - 1, i, 0]
        o_ref[pl.ds(r0, _CH), :] = _leaky(a)
        return carry

    lax.fori_loop(0, _TR // _CH, chunk, 0)


def _deep_mlp(x, w_stack):
    B, in_f = x.shape
    w_small = _rtne_bf16(w_stack[:, :_HID, :_HID])
    xt = x.T
    rows_per_step = _TR * _LANES
    pb = ((B + rows_per_step - 1) // rows_per_step) * rows_per_step
    if pb != B:
        xt = jnp.pad(xt, ((0, 0), (0, pb - B)))
    r = pb // _LANES
    xr = xt.reshape(2, r, _LANES)
    out = pl.pallas_call(
        _mlp_kernel,
        out_shape=jax.ShapeDtypeStruct((r, _LANES), jnp.float32),
        grid=(r // _TR,),
        in_specs=[
            pl.BlockSpec(memory_space=pltpu.MemorySpace.SMEM),
            pl.BlockSpec((2, _TR, _LANES), lambda b: (0, b, 0)),
        ],
        out_specs=pl.BlockSpec((_TR, _LANES), lambda b: (b, 0)),
        compiler_params=pltpu.CompilerParams(
            dimension_semantics=("parallel",),
        ),
    )(w_small, xr)
    return out.reshape(pb, 1)[:B]


def kernel(x, w_stack):
    return _deep_mlp(x, w_stack)


# speedup vs baseline: 39.0537x; 1.1500x over previous
import jax
import jax.numpy as jnp
from jax import lax
from jax.experimental import pallas as pl
from jax.experimental.pallas import tpu as pltpu

_HID = 7
_NL = 10
_SLOPE = 0.01
_LANES = 128
_TR = 512
_CH = 32


def _leaky(a):
    return jnp.maximum(a, _SLOPE * a)


def _rtne_bf16(a):
    u = lax.bitcast_convert_type(a, jnp.uint32)
    r = (u + jnp.uint32(0x7FFF) + ((u >> 16) & jnp.uint32(1))) & jnp.uint32(
        0xFFFF0000)
    return lax.bitcast_convert_type(r, jnp.float32)


def _rtna_bf16(a):
    u = lax.bitcast_convert_type(a, jnp.uint32)
    r = (u + jnp.uint32(0x8000)) & jnp.uint32(0xFFFF0000)
    return lax.bitcast_convert_type(r, jnp.float32)


def _mlp_kernel(w_ref, x_ref, o_ref):
    def chunk(c, carry):
        r0 = c * _CH
        x0 = _rtna_bf16(x_ref[0, pl.ds(r0, _CH), :])
        x1 = _rtna_bf16(x_ref[1, pl.ds(r0, _CH), :])
        h = []
        for o in range(_HID):
            a = x0 * w_ref[0, 0, o] + x1 * w_ref[0, 1, o]
            h.append(_leaky(a))
        for l in range(1, _NL - 1):
            hr = [_rtna_bf16(v) for v in h]
            hn = []
            for o in range(_HID):
                a = hr[0] * w_ref[l, 0, o]
                for i in range(1, _HID):
                    a = a + hr[i] * w_ref[l, i, o]
                hn.append(_leaky(a))
            h = hn
        hr = [_rtna_bf16(v) for v in h]
        a = hr[0] * w_ref[_NL - 1, 0, 0]
        for i in range(1, _HID):
            a = a + hr[i] * w_ref[_NL - 1, i, 0]
        o_ref[pl.ds(r0, _CH), :] = _leaky(a)
        return carry

    lax.fori_loop(0, _TR // _CH, chunk, 0)


def _deep_mlp(x, w_stack):
    B, in_f = x.shape
    w_small = _rtne_bf16(w_stack[:, :_HID, :_HID])
    xt = x.T
    rows_per_step = _TR * _LANES
    pb = ((B + rows_per_step - 1) // rows_per_step) * rows_per_step
    if pb != B:
        xt = jnp.pad(xt, ((0, 0), (0, pb - B)))
    r = pb // _LANES
    xr = xt.reshape(2, r, _LANES)
    out = pl.pallas_call(
        _mlp_kernel,
        out_shape=jax.ShapeDtypeStruct((r, _LANES), jnp.float32),
        grid=(r // _TR,),
        in_specs=[
            pl.BlockSpec(memory_space=pltpu.MemorySpace.SMEM),
            pl.BlockSpec((2, _TR, _LANES), lambda b: (0, b, 0)),
        ],
        out_specs=pl.BlockSpec((_TR, _LANES), lambda b: (b, 0)),
        compiler_params=pltpu.CompilerParams(
            dimension_semantics=("parallel",),
        ),
    )(w_small, xr)
    return out.reshape(pb, 1)[:B]


def kernel(x, w_stack):
    return _deep_mlp(x, w_stack)
